# initial kernel scaffold (unmeasured)
import functools

import jax
import jax.numpy as jnp
from jax import lax
from jax.experimental import pallas as pl
from jax.experimental.pallas import tpu as pltpu

N_DEV = 4
SQ = 2048
SKV_SHARD = 2048
HQ_LOCAL = 8
DH = 128
D_MODEL = 1024
BLK = 64
SCALE = 0.08838834764831843
QT = 1024
N_QT = SQ // QT
QUARTER = SQ // N_DEV

f32 = jnp.float32
bf16 = jnp.bfloat16


def _chunk_bias(c, t):
    qi = lax.broadcasted_iota(f32, (QT, SKV_SHARD), 0) + float(t * QT)
    ki = lax.broadcasted_iota(f32, (QT, SKV_SHARD), 1)
    qb = jnp.floor(qi * (1.0 / BLK))
    kb = jnp.floor(ki * (1.0 / BLK)) + c.astype(f32) * (SKV_SHARD // BLK)
    s = qb + kb
    mod3 = s - 3.0 * jnp.floor(s * (1.0 / 3.0))
    keep = (qb == kb) | (kb == 0.0) | (mod3 == 0.0)
    return jnp.where(keep, 0.0, -1e9)


def kernel(x, Wq, K_ext, V_ext, Wo):
    x2 = x[0].astype(bf16)
    q = jnp.dot(x2, Wq.astype(bf16), preferred_element_type=f32)
    q = q.astype(bf16)
    k_t = jnp.transpose(K_ext[0].astype(bf16), (1, 0, 2))
    v_t = jnp.transpose(V_ext[0].astype(bf16), (1, 0, 2))
    wo = Wo.astype(bf16)

    def body(
        q_ref,
        kt_ref,
        vt_ref,
        wo_ref,
        out_ref,
        k_hbm,
        v_hbm,
        k_tile,
        v_tile,
        acc_ref,
        m_ref,
        l_ref,
        bias_ref,
        rs_ref,
        ksend_sems, vsend_sems,
        krecv_sems, vrecv_sems,
        lc_sems,
        ktile_sems, vtile_sems,
        rssend_sems, rsrecv_sems,
        agsend_sems, agrecv_sems,
    ):
        me = lax.axis_index("i")

        barrier = pltpu.get_barrier_semaphore()
        for dj in range(1, N_DEV):
            pl.semaphore_signal(
                barrier, inc=1,
                device_id=((me + dj) % N_DEV,),
                device_id_type=pl.DeviceIdType.MESH,
            )
        pl.semaphore_wait(barrier, N_DEV - 1)

        sends = []
        for dj in range(1, N_DEV):
            tgt = (me + dj) % N_DEV
            for (src, dst, ssem, rsem) in (
                (kt_ref, k_hbm, ksend_sems, krecv_sems),
                (vt_ref, v_hbm, vsend_sems, vrecv_sems),
            ):
                rdma = pltpu.make_async_remote_copy(
                    src_ref=src.at[pl.ds(HQ_LOCAL * tgt, HQ_LOCAL)],
                    dst_ref=dst.at[me],
                    send_sem=ssem.at[dj - 1],
                    recv_sem=rsem.at[dj - 1],
                    device_id=(tgt,),
                    device_id_type=pl.DeviceIdType.MESH,
                )
                rdma.start()
                sends.append(rdma)

        locals_ = []
        for i, (src, dst) in enumerate(((kt_ref, k_hbm), (vt_ref, v_hbm))):
            cp = pltpu.make_async_copy(
                src.at[pl.ds(HQ_LOCAL * me, HQ_LOCAL)],
                dst.at[me],
                lc_sems.at[i],
            )
            cp.start()
            locals_.append(cp)

        for cp in locals_:
            cp.wait()
        for dj in range(1, N_DEV):
            pltpu.make_async_remote_copy(
                src_ref=kt_ref.at[pl.ds(0, HQ_LOCAL)], dst_ref=k_hbm.at[0],
                send_sem=ksend_sems.at[dj - 1], recv_sem=krecv_sems.at[dj - 1],
                device_id=(0,), device_id_type=pl.DeviceIdType.MESH,
            ).wait_recv()
            pltpu.make_async_remote_copy(
                src_ref=vt_ref.at[pl.ds(0, HQ_LOCAL)], dst_ref=v_hbm.at[0],
                send_sem=vsend_sems.at[dj - 1], recv_sem=vrecv_sems.at[dj - 1],
                device_id=(0,), device_id_type=pl.DeviceIdType.MESH,
            ).wait_recv()

        m_ref[...] = jnp.full((HQ_LOCAL, SQ, 1), -1e30, f32)
        l_ref[...] = jnp.zeros((HQ_LOCAL, SQ, 1), f32)
        acc_ref[...] = jnp.zeros((HQ_LOCAL, SQ, DH), f32)

        def tile_copies(c, h, slot):
            kc = pltpu.make_async_copy(
                k_hbm.at[c, h], k_tile.at[slot], ktile_sems.at[slot])
            vc = pltpu.make_async_copy(
                v_hbm.at[c, h], v_tile.at[slot], vtile_sems.at[slot])
            return kc, vc

        items = [(c, h) for c in range(N_DEV) for h in range(HQ_LOCAL)]
        inflight = {}
        kc, vc = tile_copies(0, 0, 0)
        kc.start(); vc.start()
        inflight[0] = (kc, vc)

        for idx, (c, h) in enumerate(items):
            if idx + 1 < len(items):
                nc, nh = items[idx + 1]
                slot = (idx + 1) % 2
                kc, vc = tile_copies(nc, nh, slot)
                kc.start(); vc.start()
                inflight[idx + 1] = (kc, vc)
            kc, vc = inflight.pop(idx)
            kc.wait(); vc.wait()
            slot = idx % 2

            if h == 0:
                cc = jnp.int32(c)
                for t in range(N_QT):
                    bias_ref[pl.ds(t * QT, QT), :] = (
                        _chunk_bias(cc, t).astype(bf16))

            k = k_tile[slot]
            v = v_tile[slot]
            for t in range(N_QT):
                qs = q_ref[pl.ds(t * QT, QT), h * DH:(h + 1) * DH]
                s = lax.dot_general(
                    qs, k, (((1,), (1,)), ((), ())),
                    preferred_element_type=f32,
                )
                s = s * SCALE + bias_ref[pl.ds(t * QT, QT), :].astype(f32)
                m_old = m_ref[h, pl.ds(t * QT, QT), :]
                m_new = jnp.maximum(m_old, jnp.max(s, axis=1, keepdims=True))
                alpha = jnp.exp(m_old - m_new)
                p = jnp.exp(s - m_new)
                l_ref[h, pl.ds(t * QT, QT), :] = (
                    l_ref[h, pl.ds(t * QT, QT), :] * alpha
                    + jnp.sum(p, axis=1, keepdims=True))
                acc_ref[h, pl.ds(t * QT, QT), :] = (
                    acc_ref[h, pl.ds(t * QT, QT), :] * alpha
                    + lax.dot_general(
                        p.astype(bf16), v, (((1,), (0,)), ((), ())),
                        preferred_element_type=f32,
                    ))
                m_ref[h, pl.ds(t * QT, QT), :] = m_new

        acc0 = (acc_ref[0] / l_ref[0]).astype(bf16)
        out_ref[...] = lax.dot_general(
            acc0, wo_ref[0:DH, :], (((1,), (0,)), ((), ())),
            preferred_element_type=f32)
        for h in range(1, HQ_LOCAL):
            ctx_h = (acc_ref[h] / l_ref[h]).astype(bf16)
            out_ref[...] += lax.dot_general(
                ctx_h, wo_ref[h * DH:(h + 1) * DH, :],
                (((1,), (0,)), ((), ())), preferred_element_type=f32)

        rs_sends = []
        for dj in range(1, N_DEV):
            tgt = (me + dj) % N_DEV
            rdma = pltpu.make_async_remote_copy(
                src_ref=out_ref.at[pl.ds(QUARTER * tgt, QUARTER), :],
                dst_ref=rs_ref.at[dj - 1],
                send_sem=rssend_sems.at[dj - 1],
                recv_sem=rsrecv_sems.at[dj - 1],
                device_id=(tgt,),
                device_id_type=pl.DeviceIdType.MESH,
            )
            rdma.start()
            rs_sends.append(rdma)
        for dj in range(1, N_DEV):
            pltpu.make_async_remote_copy(
                src_ref=out_ref.at[pl.ds(0, QUARTER), :],
                dst_ref=rs_ref.at[dj - 1],
                send_sem=rssend_sems.at[dj - 1],
                recv_sem=rsrecv_sems.at[dj - 1],
                device_id=(0,), device_id_type=pl.DeviceIdType.MESH,
            ).wait_recv()
        mine = out_ref[pl.ds(QUARTER * me, QUARTER), :]
        mine = mine + rs_ref[0] + rs_ref[1] + rs_ref[2]
        out_ref[pl.ds(QUARTER * me, QUARTER), :] = mine

        for r in rs_sends:
            r.wait_send()

        ag_sends = []
        for dj in range(1, N_DEV):
            tgt = (me + dj) % N_DEV
            rdma = pltpu.make_async_remote_copy(
                src_ref=out_ref.at[pl.ds(QUARTER * me, QUARTER), :],
                dst_ref=out_ref.at[pl.ds(QUARTER * me, QUARTER), :],
                send_sem=agsend_sems.at[dj - 1],
                recv_sem=agrecv_sems.at[dj - 1],
                device_id=(tgt,),
                device_id_type=pl.DeviceIdType.MESH,
            )
            rdma.start()
            ag_sends.append(rdma)
        for dj in range(1, N_DEV):
            pltpu.make_async_remote_copy(
                src_ref=out_ref.at[pl.ds(0, QUARTER), :],
                dst_ref=out_ref.at[pl.ds(0, QUARTER), :],
                send_sem=agsend_sems.at[dj - 1],
                recv_sem=agrecv_sems.at[dj - 1],
                device_id=(0,), device_id_type=pl.DeviceIdType.MESH,
            ).wait_recv()

        for r in sends + ag_sends:
            r.wait_send()

    out, _, _ = pl.pallas_call(
        body,
        out_shape=(
            jax.ShapeDtypeStruct((SQ, D_MODEL), f32),
            jax.ShapeDtypeStruct((N_DEV, HQ_LOCAL, SKV_SHARD, DH), bf16),
            jax.ShapeDtypeStruct((N_DEV, HQ_LOCAL, SKV_SHARD, DH), bf16),
        ),
        in_specs=[
            pl.BlockSpec(memory_space=pltpu.MemorySpace.VMEM),
            pl.BlockSpec(memory_space=pltpu.MemorySpace.ANY),
            pl.BlockSpec(memory_space=pltpu.MemorySpace.ANY),
            pl.BlockSpec(memory_space=pltpu.MemorySpace.VMEM),
        ],
        out_specs=(
            pl.BlockSpec(memory_space=pltpu.MemorySpace.VMEM),
            pl.BlockSpec(memory_space=pltpu.MemorySpace.ANY),
            pl.BlockSpec(memory_space=pltpu.MemorySpace.ANY),
        ),
        scratch_shapes=[
            pltpu.VMEM((2, SKV_SHARD, DH), bf16),
            pltpu.VMEM((2, SKV_SHARD, DH), bf16),
            pltpu.VMEM((HQ_LOCAL, SQ, DH), f32),
            pltpu.VMEM((HQ_LOCAL, SQ, 1), f32),
            pltpu.VMEM((HQ_LOCAL, SQ, 1), f32),
            pltpu.VMEM((SQ, SKV_SHARD), bf16),
            pltpu.VMEM((3, QUARTER, D_MODEL), f32),
            pltpu.SemaphoreType.DMA((3,)),
            pltpu.SemaphoreType.DMA((3,)),
            pltpu.SemaphoreType.DMA((3,)),
            pltpu.SemaphoreType.DMA((3,)),
            pltpu.SemaphoreType.DMA((2,)),
            pltpu.SemaphoreType.DMA((2,)),
            pltpu.SemaphoreType.DMA((2,)),
            pltpu.SemaphoreType.DMA((3,)),
            pltpu.SemaphoreType.DMA((3,)),
            pltpu.SemaphoreType.DMA((3,)),
            pltpu.SemaphoreType.DMA((3,)),
        ],
        compiler_params=pltpu.CompilerParams(
            collective_id=0,
            vmem_limit_bytes=64 * 1024 * 1024,
        ),
    )(q, k_t, v_t, wo)

    return out[None]


# baseline (device time: 709845 ns/iter reference)
import functools

import jax
import jax.numpy as jnp
from jax import lax
from jax.experimental import pallas as pl
from jax.experimental.pallas import tpu as pltpu

N_DEV = 4
SQ = 2048
SKV_SHARD = 2048
HQ_LOCAL = 8
DH = 128
D_MODEL = 1024
BLK = 64
SCALE = 0.08838834764831843
QT = 512
N_QT = SQ // QT
QUARTER = SQ // N_DEV

f32 = jnp.float32
bf16 = jnp.bfloat16


def _mask_tile(c, t):
    toff = (t * QT).astype(f32)
    qi = lax.broadcasted_iota(jnp.int32, (QT, SKV_SHARD), 0).astype(f32) + toff
    ki = lax.broadcasted_iota(jnp.int32, (QT, SKV_SHARD), 1).astype(f32)
    qb = jnp.floor(qi * (1.0 / BLK))
    kb = jnp.floor(ki * (1.0 / BLK)) + float(c * (SKV_SHARD // BLK))
    sm = qb + kb
    mod3 = sm - 3.0 * jnp.floor(sm * (1.0 / 3.0))
    keep = (qb == kb) | (kb == 0.0) | (mod3 == 0.0)
    return keep


def kernel(x, Wq, K_ext, V_ext, Wo):
    x2 = x[0].astype(bf16)
    q = jnp.dot(x2, Wq.astype(bf16), preferred_element_type=f32)
    q = q.astype(bf16)
    k_t = jnp.transpose(K_ext[0].astype(bf16), (1, 0, 2))
    v_t = jnp.transpose(V_ext[0].astype(bf16), (1, 0, 2))
    wo = Wo.astype(bf16)

    def body(
        q_ref,
        kt_ref,
        vt_ref,
        wo_ref,
        out_ref,
        k_hbm,
        v_hbm,
        k_tile,
        v_tile,
        acc_ref,
        l_ref,
        mask_ref,
        rs_ref,
        ksend_sems, vsend_sems,
        krecv_sems, vrecv_sems,
        lc_sems,
        ktile_sems, vtile_sems,
        rssend_sems, rsrecv_sems,
        agsend_sems, agrecv_sems,
    ):
        me = lax.axis_index("i")

        barrier = pltpu.get_barrier_semaphore()
        for dj in range(1, N_DEV):
            pl.semaphore_signal(
                barrier, inc=1,
                device_id=((me + dj) % N_DEV,),
                device_id_type=pl.DeviceIdType.MESH,
            )
        pl.semaphore_wait(barrier, N_DEV - 1)

        sends = []
        for dj in range(1, N_DEV):
            tgt = (me + dj) % N_DEV
            for (src, dst, ssem, rsem) in (
                (kt_ref, k_hbm, ksend_sems, krecv_sems),
                (vt_ref, v_hbm, vsend_sems, vrecv_sems),
            ):
                rdma = pltpu.make_async_remote_copy(
                    src_ref=src.at[pl.ds(HQ_LOCAL * tgt, HQ_LOCAL)],
                    dst_ref=dst.at[me],
                    send_sem=ssem.at[dj - 1],
                    recv_sem=rsem.at[dj - 1],
                    device_id=(tgt,),
                    device_id_type=pl.DeviceIdType.MESH,
                )
                rdma.start()
                sends.append(rdma)

        locals_ = []
        for i, (src, dst) in enumerate(((kt_ref, k_hbm), (vt_ref, v_hbm))):
            cp = pltpu.make_async_copy(
                src.at[pl.ds(HQ_LOCAL * me, HQ_LOCAL)],
                dst.at[me],
                lc_sems.at[i],
            )
            cp.start()
            locals_.append(cp)

        for cp in locals_:
            cp.wait()
        for dj in range(1, N_DEV):
            pltpu.make_async_remote_copy(
                src_ref=kt_ref.at[pl.ds(0, HQ_LOCAL)], dst_ref=k_hbm.at[0],
                send_sem=ksend_sems.at[dj - 1], recv_sem=krecv_sems.at[dj - 1],
                device_id=(0,), device_id_type=pl.DeviceIdType.MESH,
            ).wait_recv()
            pltpu.make_async_remote_copy(
                src_ref=vt_ref.at[pl.ds(0, HQ_LOCAL)], dst_ref=v_hbm.at[0],
                send_sem=vsend_sems.at[dj - 1], recv_sem=vrecv_sems.at[dj - 1],
                device_id=(0,), device_id_type=pl.DeviceIdType.MESH,
            ).wait_recv()

        l_ref[...] = jnp.zeros((HQ_LOCAL, SQ, DH), f32)
        acc_ref[...] = jnp.zeros((HQ_LOCAL, SQ, DH), f32)

        def tile_copies(c, h, slot):
            kc = pltpu.make_async_copy(
                k_hbm.at[c, h], k_tile.at[slot], ktile_sems.at[slot])
            vc = pltpu.make_async_copy(
                v_hbm.at[c, h], v_tile.at[slot], vtile_sems.at[slot])
            return kc, vc

        items = [(c, h) for c in range(N_DEV) for h in range(HQ_LOCAL)]
        inflight = {}
        kc, vc = tile_copies(0, 0, 0)
        kc.start(); vc.start()
        inflight[0] = (kc, vc)

        for idx, (c, h) in enumerate(items):
            if idx + 1 < len(items):
                nc, nh = items[idx + 1]
                slot = (idx + 1) % 2
                kc, vc = tile_copies(nc, nh, slot)
                kc.start(); vc.start()
                inflight[idx + 1] = (kc, vc)
            kc, vc = inflight.pop(idx)
            kc.wait(); vc.wait()
            slot = idx % 2

            if h == 0:
                def mask_body(t, _):
                    mask_ref[pl.ds(t * QT, QT), :] = (
                        _mask_tile(c, t).astype(jnp.int8))
                    return _
                lax.fori_loop(0, N_QT, mask_body, None, unroll=False)

            k = k_tile[slot]
            v = v_tile[slot]

            def tile_body(t, _):
                rows = pl.ds(t * QT, QT)
                s = lax.dot_general(
                    q_ref[rows, h * DH:(h + 1) * DH],
                    k, (((1,), (1,)), ((), ())),
                    preferred_element_type=f32,
                )
                p = jnp.exp(s * SCALE) * mask_ref[rows, :].astype(f32)
                l_ref[h, rows, :] += jnp.sum(p, axis=1, keepdims=True)
                acc_ref[h, rows, :] += lax.dot_general(
                    p.astype(bf16), v, (((1,), (0,)), ((), ())),
                    preferred_element_type=f32,
                )
                return _
            lax.fori_loop(0, N_QT, tile_body, None, unroll=False)

        acc0 = (acc_ref[0] / l_ref[0]).astype(bf16)
        out_ref[...] = lax.dot_general(
            acc0, wo_ref[0:DH, :], (((1,), (0,)), ((), ())),
            preferred_element_type=f32)
        for h in range(1, HQ_LOCAL):
            ctx_h = (acc_ref[h] / l_ref[h]).astype(bf16)
            out_ref[...] += lax.dot_general(
                ctx_h, wo_ref[h * DH:(h + 1) * DH, :],
                (((1,), (0,)), ((), ())), preferred_element_type=f32)

        rs_sends = []
        for dj in range(1, N_DEV):
            tgt = (me + dj) % N_DEV
            rdma = pltpu.make_async_remote_copy(
                src_ref=out_ref.at[pl.ds(QUARTER * tgt, QUARTER), :],
                dst_ref=rs_ref.at[dj - 1],
                send_sem=rssend_sems.at[dj - 1],
                recv_sem=rsrecv_sems.at[dj - 1],
                device_id=(tgt,),
                device_id_type=pl.DeviceIdType.MESH,
            )
            rdma.start()
            rs_sends.append(rdma)
        for dj in range(1, N_DEV):
            pltpu.make_async_remote_copy(
                src_ref=out_ref.at[pl.ds(0, QUARTER), :],
                dst_ref=rs_ref.at[dj - 1],
                send_sem=rssend_sems.at[dj - 1],
                recv_sem=rsrecv_sems.at[dj - 1],
                device_id=(0,), device_id_type=pl.DeviceIdType.MESH,
            ).wait_recv()
        mine = out_ref[pl.ds(QUARTER * me, QUARTER), :]
        mine = mine + rs_ref[0] + rs_ref[1] + rs_ref[2]
        out_ref[pl.ds(QUARTER * me, QUARTER), :] = mine

        for r in rs_sends:
            r.wait_send()

        ag_sends = []
        for dj in range(1, N_DEV):
            tgt = (me + dj) % N_DEV
            rdma = pltpu.make_async_remote_copy(
                src_ref=out_ref.at[pl.ds(QUARTER * me, QUARTER), :],
                dst_ref=out_ref.at[pl.ds(QUARTER * me, QUARTER), :],
                send_sem=agsend_sems.at[dj - 1],
                recv_sem=agrecv_sems.at[dj - 1],
                device_id=(tgt,),
                device_id_type=pl.DeviceIdType.MESH,
            )
            rdma.start()
            ag_sends.append(rdma)
        for dj in range(1, N_DEV):
            pltpu.make_async_remote_copy(
                src_ref=out_ref.at[pl.ds(0, QUARTER), :],
                dst_ref=out_ref.at[pl.ds(0, QUARTER), :],
                send_sem=agsend_sems.at[dj - 1],
                recv_sem=agrecv_sems.at[dj - 1],
                device_id=(0,), device_id_type=pl.DeviceIdType.MESH,
            ).wait_recv()

        for r in sends + ag_sends:
            r.wait_send()

    out, _, _ = pl.pallas_call(
        body,
        out_shape=(
            jax.ShapeDtypeStruct((SQ, D_MODEL), f32),
            jax.ShapeDtypeStruct((N_DEV, HQ_LOCAL, SKV_SHARD, DH), bf16),
            jax.ShapeDtypeStruct((N_DEV, HQ_LOCAL, SKV_SHARD, DH), bf16),
        ),
        in_specs=[
            pl.BlockSpec(memory_space=pltpu.MemorySpace.VMEM),
            pl.BlockSpec(memory_space=pl.ANY),
            pl.BlockSpec(memory_space=pl.ANY),
            pl.BlockSpec(memory_space=pltpu.MemorySpace.VMEM),
        ],
        out_specs=(
            pl.BlockSpec(memory_space=pltpu.MemorySpace.VMEM),
            pl.BlockSpec(memory_space=pl.ANY),
            pl.BlockSpec(memory_space=pl.ANY),
        ),
        scratch_shapes=[
            pltpu.VMEM((2, SKV_SHARD, DH), bf16),
            pltpu.VMEM((2, SKV_SHARD, DH), bf16),
            pltpu.VMEM((HQ_LOCAL, SQ, DH), f32),
            pltpu.VMEM((HQ_LOCAL, SQ, DH), f32),
            pltpu.VMEM((SQ, SKV_SHARD), jnp.int8),
            pltpu.VMEM((3, QUARTER, D_MODEL), f32),
            pltpu.SemaphoreType.DMA((3,)),
            pltpu.SemaphoreType.DMA((3,)),
            pltpu.SemaphoreType.DMA((3,)),
            pltpu.SemaphoreType.DMA((3,)),
            pltpu.SemaphoreType.DMA((2,)),
            pltpu.SemaphoreType.DMA((2,)),
            pltpu.SemaphoreType.DMA((2,)),
            pltpu.SemaphoreType.DMA((3,)),
            pltpu.SemaphoreType.DMA((3,)),
            pltpu.SemaphoreType.DMA((3,)),
            pltpu.SemaphoreType.DMA((3,)),
        ],
        compiler_params=pltpu.CompilerParams(
            collective_id=0,
            vmem_limit_bytes=64 * 1024 * 1024,
        ),
    )(q, k_t, v_t, wo)

    return out[None]


# device time: 663602 ns/iter; 1.0697x vs baseline; 1.0697x over previous
import functools

import jax
import jax.numpy as jnp
from jax import lax
from jax.experimental import pallas as pl
from jax.experimental.pallas import tpu as pltpu

N_DEV = 4
SQ = 2048
SKV_SHARD = 2048
HQ_LOCAL = 8
DH = 128
D_MODEL = 1024
BLK = 64
SCALE = 0.08838834764831843
QT = 512
N_QT = SQ // QT
QUARTER = SQ // N_DEV

f32 = jnp.float32
bf16 = jnp.bfloat16


def _mask_tile(coff, t):
    toff = (t * QT).astype(f32)
    qi = lax.broadcasted_iota(jnp.int32, (QT, SKV_SHARD), 0).astype(f32) + toff
    ki = lax.broadcasted_iota(jnp.int32, (QT, SKV_SHARD), 1).astype(f32)
    qb = jnp.floor(qi * (1.0 / BLK))
    kb = jnp.floor(ki * (1.0 / BLK)) + coff
    sm = qb + kb
    mod3 = sm - 3.0 * jnp.floor(sm * (1.0 / 3.0))
    keep = (qb == kb) | (kb == 0.0) | (mod3 == 0.0)
    return keep


def kernel(x, Wq, K_ext, V_ext, Wo):
    x2 = x[0].astype(bf16)
    q = jnp.dot(x2, Wq.astype(bf16), preferred_element_type=f32)
    q = q.astype(bf16)
    k_t = jnp.transpose(K_ext[0].astype(bf16), (1, 0, 2))
    v_t = jnp.transpose(V_ext[0].astype(bf16), (1, 0, 2))
    wo = Wo.astype(bf16)

    def body(
        q_ref,
        kt_ref,
        vt_ref,
        wo_ref,
        out_ref,
        k_hbm,
        v_hbm,
        k_tile,
        v_tile,
        acc_ref,
        l_ref,
        mask_ref,
        rs_ref,
        pb_ref,
        agq_ref,
        agr_ref,
        ksend_sems, vsend_sems,
        krecv_sems, vrecv_sems,
        lc_sems,
        ktile_sems, vtile_sems,
        rssend_sems, rsrecv_sems,
        agsend_sems, agrecv_sems,
    ):
        me = lax.axis_index("i")

        barrier = pltpu.get_barrier_semaphore()
        for dj in range(1, N_DEV):
            pl.semaphore_signal(
                barrier, inc=1,
                device_id=((me + dj) % N_DEV,),
                device_id_type=pl.DeviceIdType.MESH,
            )
        pl.semaphore_wait(barrier, N_DEV - 1)

        sends = []
        for dj in range(1, N_DEV):
            tgt = (me + dj) % N_DEV
            for (src, dst, ssem, rsem) in (
                (kt_ref, k_hbm, ksend_sems, krecv_sems),
                (vt_ref, v_hbm, vsend_sems, vrecv_sems),
            ):
                rdma = pltpu.make_async_remote_copy(
                    src_ref=src.at[pl.ds(HQ_LOCAL * tgt, HQ_LOCAL)],
                    dst_ref=dst.at[me],
                    send_sem=ssem.at[dj - 1],
                    recv_sem=rsem.at[dj - 1],
                    device_id=(tgt,),
                    device_id_type=pl.DeviceIdType.MESH,
                )
                rdma.start()
                sends.append(rdma)

        locals_ = []
        for i, (src, dst) in enumerate(((kt_ref, k_hbm), (vt_ref, v_hbm))):
            cp = pltpu.make_async_copy(
                src.at[pl.ds(HQ_LOCAL * me, HQ_LOCAL)],
                dst.at[me],
                lc_sems.at[i],
            )
            cp.start()
            locals_.append(cp)

        def wait_chunk(dc):
            if dc == 0:
                for cp in locals_:
                    cp.wait()
                return
            sl = 3 - dc
            pltpu.make_async_remote_copy(
                src_ref=kt_ref.at[pl.ds(0, HQ_LOCAL)], dst_ref=k_hbm.at[0],
                send_sem=ksend_sems.at[sl], recv_sem=krecv_sems.at[sl],
                device_id=(0,), device_id_type=pl.DeviceIdType.MESH,
            ).wait_recv()
            pltpu.make_async_remote_copy(
                src_ref=vt_ref.at[pl.ds(0, HQ_LOCAL)], dst_ref=v_hbm.at[0],
                send_sem=vsend_sems.at[sl], recv_sem=vrecv_sems.at[sl],
                device_id=(0,), device_id_type=pl.DeviceIdType.MESH,
            ).wait_recv()

        l_ref[...] = jnp.zeros((HQ_LOCAL, SQ, DH), f32)
        acc_ref[...] = jnp.zeros((HQ_LOCAL, SQ, DH), f32)

        def tile_copies(cv, h, slot):
            kc = pltpu.make_async_copy(
                k_hbm.at[cv, h], k_tile.at[slot], ktile_sems.at[slot])
            vc = pltpu.make_async_copy(
                v_hbm.at[cv, h], v_tile.at[slot], vtile_sems.at[slot])
            return kc, vc

        DCS = [0, 1, 3, 2]
        cvals = [(me + dc) % N_DEV for dc in DCS]
        items = [(ci, h) for ci in range(N_DEV) for h in range(HQ_LOCAL)]
        inflight = {}
        wait_chunk(DCS[0])
        kc, vc = tile_copies(cvals[0], 0, 0)
        kc.start(); vc.start()
        inflight[0] = (kc, vc)

        for idx, (ci, h) in enumerate(items):
            if idx + 1 < len(items):
                nci, nh = items[idx + 1]
                if nh == 0:
                    wait_chunk(DCS[nci])
                slot = (idx + 1) % 2
                kc, vc = tile_copies(cvals[nci], nh, slot)
                kc.start(); vc.start()
                inflight[idx + 1] = (kc, vc)
            kc, vc = inflight.pop(idx)
            kc.wait(); vc.wait()
            slot = idx % 2

            if h == 0:
                coff = (cvals[ci] * (SKV_SHARD // BLK)).astype(f32)

                def mask_body(t, _, coff=coff):
                    mask_ref[pl.ds(t * QT, QT), :] = (
                        _mask_tile(coff, t).astype(jnp.int8))
                    return _
                lax.fori_loop(0, N_QT, mask_body, None, unroll=False)

            k = k_tile[slot]
            v = v_tile[slot]

            def tile_body(t, _):
                rows = pl.ds(t * QT, QT)
                s = lax.dot_general(
                    q_ref[rows, h * DH:(h + 1) * DH],
                    k, (((1,), (1,)), ((), ())),
                    preferred_element_type=f32,
                )
                p = jnp.exp(s * SCALE) * mask_ref[rows, :].astype(f32)
                l_ref[h, rows, :] += jnp.sum(p, axis=1, keepdims=True)
                acc_ref[h, rows, :] += lax.dot_general(
                    p.astype(bf16), v, (((1,), (0,)), ((), ())),
                    preferred_element_type=f32,
                )
                return _
            lax.fori_loop(0, N_QT, tile_body, None, unroll=False)

        acc0 = (acc_ref[0] / l_ref[0]).astype(bf16)
        out_ref[...] = lax.dot_general(
            acc0, wo_ref[0:DH, :], (((1,), (0,)), ((), ())),
            preferred_element_type=f32)
        for h in range(1, HQ_LOCAL):
            ctx_h = (acc_ref[h] / l_ref[h]).astype(bf16)
            out_ref[...] += lax.dot_general(
                ctx_h, wo_ref[h * DH:(h + 1) * DH, :],
                (((1,), (0,)), ((), ())), preferred_element_type=f32)

        pb_ref[...] = out_ref[...].astype(bf16)
        rs_sends = []
        for dj in range(1, N_DEV):
            tgt = (me + dj) % N_DEV
            rdma = pltpu.make_async_remote_copy(
                src_ref=pb_ref.at[pl.ds(QUARTER * tgt, QUARTER), :],
                dst_ref=rs_ref.at[dj - 1],
                send_sem=rssend_sems.at[dj - 1],
                recv_sem=rsrecv_sems.at[dj - 1],
                device_id=(tgt,),
                device_id_type=pl.DeviceIdType.MESH,
            )
            rdma.start()
            rs_sends.append(rdma)
        for dj in range(1, N_DEV):
            pltpu.make_async_remote_copy(
                src_ref=pb_ref.at[pl.ds(0, QUARTER), :],
                dst_ref=rs_ref.at[dj - 1],
                send_sem=rssend_sems.at[dj - 1],
                recv_sem=rsrecv_sems.at[dj - 1],
                device_id=(0,), device_id_type=pl.DeviceIdType.MESH,
            ).wait_recv()
        mine = out_ref[pl.ds(QUARTER * me, QUARTER), :]
        mine = (mine + rs_ref[0].astype(f32) + rs_ref[1].astype(f32)
                + rs_ref[2].astype(f32))
        out_ref[pl.ds(QUARTER * me, QUARTER), :] = mine
        agq_ref[...] = mine.astype(bf16)
        for r in rs_sends:
            r.wait_send()

        ag_sends = []
        for dj in range(1, N_DEV):
            tgt = (me + dj) % N_DEV
            rdma = pltpu.make_async_remote_copy(
                src_ref=agq_ref,
                dst_ref=agr_ref.at[dj - 1],
                send_sem=agsend_sems.at[dj - 1],
                recv_sem=agrecv_sems.at[dj - 1],
                device_id=(tgt,),
                device_id_type=pl.DeviceIdType.MESH,
            )
            rdma.start()
            ag_sends.append(rdma)
        for dj in range(1, N_DEV):
            pltpu.make_async_remote_copy(
                src_ref=agq_ref,
                dst_ref=agr_ref.at[dj - 1],
                send_sem=agsend_sems.at[dj - 1],
                recv_sem=agrecv_sems.at[dj - 1],
                device_id=(0,), device_id_type=pl.DeviceIdType.MESH,
            ).wait_recv()
        for sl in range(N_DEV - 1):
            src_dev = (me - sl - 1) % N_DEV
            out_ref[pl.ds(QUARTER * src_dev, QUARTER), :] = (
                agr_ref[sl].astype(f32))

        for r in sends + ag_sends:
            r.wait_send()

    out, _, _ = pl.pallas_call(
        body,
        out_shape=(
            jax.ShapeDtypeStruct((SQ, D_MODEL), f32),
            jax.ShapeDtypeStruct((N_DEV, HQ_LOCAL, SKV_SHARD, DH), bf16),
            jax.ShapeDtypeStruct((N_DEV, HQ_LOCAL, SKV_SHARD, DH), bf16),
        ),
        in_specs=[
            pl.BlockSpec(memory_space=pltpu.MemorySpace.VMEM),
            pl.BlockSpec(memory_space=pl.ANY),
            pl.BlockSpec(memory_space=pl.ANY),
            pl.BlockSpec(memory_space=pltpu.MemorySpace.VMEM),
        ],
        out_specs=(
            pl.BlockSpec(memory_space=pltpu.MemorySpace.VMEM),
            pl.BlockSpec(memory_space=pl.ANY),
            pl.BlockSpec(memory_space=pl.ANY),
        ),
        scratch_shapes=[
            pltpu.VMEM((2, SKV_SHARD, DH), bf16),
            pltpu.VMEM((2, SKV_SHARD, DH), bf16),
            pltpu.VMEM((HQ_LOCAL, SQ, DH), f32),
            pltpu.VMEM((HQ_LOCAL, SQ, DH), f32),
            pltpu.VMEM((SQ, SKV_SHARD), jnp.int8),
            pltpu.VMEM((3, QUARTER, D_MODEL), bf16),
            pltpu.VMEM((SQ, D_MODEL), bf16),
            pltpu.VMEM((QUARTER, D_MODEL), bf16),
            pltpu.VMEM((3, QUARTER, D_MODEL), bf16),
            pltpu.SemaphoreType.DMA((3,)),
            pltpu.SemaphoreType.DMA((3,)),
            pltpu.SemaphoreType.DMA((3,)),
            pltpu.SemaphoreType.DMA((3,)),
            pltpu.SemaphoreType.DMA((2,)),
            pltpu.SemaphoreType.DMA((2,)),
            pltpu.SemaphoreType.DMA((2,)),
            pltpu.SemaphoreType.DMA((3,)),
            pltpu.SemaphoreType.DMA((3,)),
            pltpu.SemaphoreType.DMA((3,)),
            pltpu.SemaphoreType.DMA((3,)),
        ],
        compiler_params=pltpu.CompilerParams(
            collective_id=0,
            vmem_limit_bytes=64 * 1024 * 1024,
        ),
    )(q, k_t, v_t, wo)

    return out[None]


# device time: 637827 ns/iter; 1.1129x vs baseline; 1.0404x over previous
import functools

import jax
import jax.numpy as jnp
from jax import lax
from jax.experimental import pallas as pl
from jax.experimental.pallas import tpu as pltpu

N_DEV = 4
SQ = 2048
SKV_SHARD = 2048
HQ_LOCAL = 8
DH = 128
D_MODEL = 1024
BLK = 64
SCALE = 0.08838834764831843
QT = 512
N_QT = SQ // QT
QUARTER = SQ // N_DEV

f32 = jnp.float32
bf16 = jnp.bfloat16


def _mask_tile(coff, t):
    toff = (t * QT).astype(f32)
    qi = lax.broadcasted_iota(jnp.int32, (QT, SKV_SHARD), 0).astype(f32) + toff
    ki = lax.broadcasted_iota(jnp.int32, (QT, SKV_SHARD), 1).astype(f32)
    qb = jnp.floor(qi * (1.0 / BLK))
    kb = jnp.floor(ki * (1.0 / BLK)) + coff
    sm = qb + kb
    mod3 = sm - 3.0 * jnp.floor(sm * (1.0 / 3.0))
    keep = (qb == kb) | (kb == 0.0) | (mod3 == 0.0)
    return keep


def kernel(x, Wq, K_ext, V_ext, Wo):
    x2 = x[0].astype(bf16)
    q = jnp.dot(x2, Wq.astype(bf16), preferred_element_type=f32)
    q = (q * SCALE).astype(bf16)
    k_t = jnp.transpose(K_ext[0].astype(bf16), (1, 0, 2))
    v_t = jnp.transpose(V_ext[0].astype(bf16), (1, 0, 2))
    wo = Wo.astype(bf16)

    def body(
        q_ref,
        kt_ref,
        vt_ref,
        wo_ref,
        out_ref,
        k_hbm,
        v_hbm,
        k_tile,
        v_tile,
        acc_ref,
        l_ref,
        mask_ref,
        rs_ref,
        pb_ref,
        agq_ref,
        agr_ref,
        ksend_sems, vsend_sems,
        krecv_sems, vrecv_sems,
        lc_sems,
        ktile_sems, vtile_sems,
        rssend_sems, rsrecv_sems,
        agsend_sems, agrecv_sems,
    ):
        me = lax.axis_index("i")

        barrier = pltpu.get_barrier_semaphore()
        for dj in range(1, N_DEV):
            pl.semaphore_signal(
                barrier, inc=1,
                device_id=((me + dj) % N_DEV,),
                device_id_type=pl.DeviceIdType.MESH,
            )
        pl.semaphore_wait(barrier, N_DEV - 1)

        sends = []
        for dj in range(1, N_DEV):
            tgt = (me + dj) % N_DEV
            for (src, dst, ssem, rsem) in (
                (kt_ref, k_hbm, ksend_sems, krecv_sems),
                (vt_ref, v_hbm, vsend_sems, vrecv_sems),
            ):
                rdma = pltpu.make_async_remote_copy(
                    src_ref=src.at[pl.ds(HQ_LOCAL * tgt, HQ_LOCAL)],
                    dst_ref=dst.at[me],
                    send_sem=ssem.at[dj - 1],
                    recv_sem=rsem.at[dj - 1],
                    device_id=(tgt,),
                    device_id_type=pl.DeviceIdType.MESH,
                )
                rdma.start()
                sends.append(rdma)

        locals_ = []
        for i, (src, dst) in enumerate(((kt_ref, k_hbm), (vt_ref, v_hbm))):
            cp = pltpu.make_async_copy(
                src.at[pl.ds(HQ_LOCAL * me, HQ_LOCAL)],
                dst.at[me],
                lc_sems.at[i],
            )
            cp.start()
            locals_.append(cp)

        def wait_chunk(dc):
            if dc == 0:
                for cp in locals_:
                    cp.wait()
                return
            sl = 3 - dc
            pltpu.make_async_remote_copy(
                src_ref=kt_ref.at[pl.ds(0, HQ_LOCAL)], dst_ref=k_hbm.at[0],
                send_sem=ksend_sems.at[sl], recv_sem=krecv_sems.at[sl],
                device_id=(0,), device_id_type=pl.DeviceIdType.MESH,
            ).wait_recv()
            pltpu.make_async_remote_copy(
                src_ref=vt_ref.at[pl.ds(0, HQ_LOCAL)], dst_ref=v_hbm.at[0],
                send_sem=vsend_sems.at[sl], recv_sem=vrecv_sems.at[sl],
                device_id=(0,), device_id_type=pl.DeviceIdType.MESH,
            ).wait_recv()

        l_ref[...] = jnp.zeros((HQ_LOCAL, SQ, DH), f32)
        acc_ref[...] = jnp.zeros((HQ_LOCAL, SQ, DH), f32)

        def tile_copies(cv, h, slot):
            kc = pltpu.make_async_copy(
                k_hbm.at[cv, h], k_tile.at[slot], ktile_sems.at[slot])
            vc = pltpu.make_async_copy(
                v_hbm.at[cv, h], v_tile.at[slot], vtile_sems.at[slot])
            return kc, vc

        DCS = [0, 1, 3, 2]
        cvals = [(me + dc) % N_DEV for dc in DCS]
        items = [(ci, h) for ci in range(N_DEV) for h in range(HQ_LOCAL)]
        inflight = {}
        wait_chunk(DCS[0])
        kc, vc = tile_copies(cvals[0], 0, 0)
        kc.start(); vc.start()
        inflight[0] = (kc, vc)

        for idx, (ci, h) in enumerate(items):
            if idx + 1 < len(items):
                nci, nh = items[idx + 1]
                if nh == 0:
                    wait_chunk(DCS[nci])
                slot = (idx + 1) % 2
                kc, vc = tile_copies(cvals[nci], nh, slot)
                kc.start(); vc.start()
                inflight[idx + 1] = (kc, vc)
            kc, vc = inflight.pop(idx)
            kc.wait(); vc.wait()
            slot = idx % 2

            if h == 0:
                coff = (cvals[ci] * (SKV_SHARD // BLK)).astype(f32)

                def mask_body(t, _, coff=coff):
                    mask_ref[pl.ds(t * QT, QT), :] = (
                        _mask_tile(coff, t).astype(jnp.int8))
                    return _
                lax.fori_loop(0, N_QT, mask_body, None, unroll=False)

            k = k_tile[slot]
            v = v_tile[slot]

            def tile_body(t, _):
                rows = pl.ds(t * QT, QT)
                s = lax.dot_general(
                    q_ref[rows, h * DH:(h + 1) * DH],
                    k, (((1,), (1,)), ((), ())),
                    preferred_element_type=f32,
                )
                p = jnp.exp(s) * mask_ref[rows, :].astype(f32)
                l_ref[h, rows, :] += jnp.sum(p, axis=1, keepdims=True)
                acc_ref[h, rows, :] += lax.dot_general(
                    p.astype(bf16), v, (((1,), (0,)), ((), ())),
                    preferred_element_type=f32,
                )
                return _
            lax.fori_loop(0, N_QT, tile_body, None, unroll=False)

        rs_sends = []
        for dj in range(1, N_DEV + 1):
            tgt = (me + dj) % N_DEV
            rows = pl.ds(QUARTER * tgt, QUARTER)
            part = jnp.zeros((QUARTER, D_MODEL), f32)
            for h in range(HQ_LOCAL):
                ctx_h = (acc_ref[h, rows, :] / l_ref[h, rows, :]).astype(bf16)
                part = part + lax.dot_general(
                    ctx_h, wo_ref[h * DH:(h + 1) * DH, :],
                    (((1,), (0,)), ((), ())), preferred_element_type=f32)
            out_ref[rows, :] = part
            if dj < N_DEV:
                pb_ref[rows, :] = part.astype(bf16)
                rdma = pltpu.make_async_remote_copy(
                    src_ref=pb_ref.at[rows, :],
                    dst_ref=rs_ref.at[dj - 1],
                    send_sem=rssend_sems.at[dj - 1],
                    recv_sem=rsrecv_sems.at[dj - 1],
                    device_id=(tgt,),
                    device_id_type=pl.DeviceIdType.MESH,
                )
                rdma.start()
                rs_sends.append(rdma)
        for dj in range(1, N_DEV):
            pltpu.make_async_remote_copy(
                src_ref=pb_ref.at[pl.ds(0, QUARTER), :],
                dst_ref=rs_ref.at[dj - 1],
                send_sem=rssend_sems.at[dj - 1],
                recv_sem=rsrecv_sems.at[dj - 1],
                device_id=(0,), device_id_type=pl.DeviceIdType.MESH,
            ).wait_recv()
        mine = out_ref[pl.ds(QUARTER * me, QUARTER), :]
        mine = (mine + rs_ref[0].astype(f32) + rs_ref[1].astype(f32)
                + rs_ref[2].astype(f32))
        out_ref[pl.ds(QUARTER * me, QUARTER), :] = mine
        agq_ref[...] = mine.astype(bf16)
        for r in rs_sends:
            r.wait_send()

        ag_sends = []
        for dj in range(1, N_DEV):
            tgt = (me + dj) % N_DEV
            rdma = pltpu.make_async_remote_copy(
                src_ref=agq_ref,
                dst_ref=agr_ref.at[dj - 1],
                send_sem=agsend_sems.at[dj - 1],
                recv_sem=agrecv_sems.at[dj - 1],
                device_id=(tgt,),
                device_id_type=pl.DeviceIdType.MESH,
            )
            rdma.start()
            ag_sends.append(rdma)
        for dj in range(1, N_DEV):
            pltpu.make_async_remote_copy(
                src_ref=agq_ref,
                dst_ref=agr_ref.at[dj - 1],
                send_sem=agsend_sems.at[dj - 1],
                recv_sem=agrecv_sems.at[dj - 1],
                device_id=(0,), device_id_type=pl.DeviceIdType.MESH,
            ).wait_recv()
        for sl in range(N_DEV - 1):
            src_dev = (me - sl - 1) % N_DEV
            out_ref[pl.ds(QUARTER * src_dev, QUARTER), :] = (
                agr_ref[sl].astype(f32))

        for r in sends + ag_sends:
            r.wait_send()

    out, _, _ = pl.pallas_call(
        body,
        out_shape=(
            jax.ShapeDtypeStruct((SQ, D_MODEL), f32),
            jax.ShapeDtypeStruct((N_DEV, HQ_LOCAL, SKV_SHARD, DH), bf16),
            jax.ShapeDtypeStruct((N_DEV, HQ_LOCAL, SKV_SHARD, DH), bf16),
        ),
        in_specs=[
            pl.BlockSpec(memory_space=pltpu.MemorySpace.VMEM),
            pl.BlockSpec(memory_space=pl.ANY),
            pl.BlockSpec(memory_space=pl.ANY),
            pl.BlockSpec(memory_space=pltpu.MemorySpace.VMEM),
        ],
        out_specs=(
            pl.BlockSpec(memory_space=pltpu.MemorySpace.VMEM),
            pl.BlockSpec(memory_space=pl.ANY),
            pl.BlockSpec(memory_space=pl.ANY),
        ),
        scratch_shapes=[
            pltpu.VMEM((2, SKV_SHARD, DH), bf16),
            pltpu.VMEM((2, SKV_SHARD, DH), bf16),
            pltpu.VMEM((HQ_LOCAL, SQ, DH), f32),
            pltpu.VMEM((HQ_LOCAL, SQ, DH), f32),
            pltpu.VMEM((SQ, SKV_SHARD), jnp.int8),
            pltpu.VMEM((3, QUARTER, D_MODEL), bf16),
            pltpu.VMEM((SQ, D_MODEL), bf16),
            pltpu.VMEM((QUARTER, D_MODEL), bf16),
            pltpu.VMEM((3, QUARTER, D_MODEL), bf16),
            pltpu.SemaphoreType.DMA((3,)),
            pltpu.SemaphoreType.DMA((3,)),
            pltpu.SemaphoreType.DMA((3,)),
            pltpu.SemaphoreType.DMA((3,)),
            pltpu.SemaphoreType.DMA((2,)),
            pltpu.SemaphoreType.DMA((2,)),
            pltpu.SemaphoreType.DMA((2,)),
            pltpu.SemaphoreType.DMA((3,)),
            pltpu.SemaphoreType.DMA((3,)),
            pltpu.SemaphoreType.DMA((3,)),
            pltpu.SemaphoreType.DMA((3,)),
        ],
        compiler_params=pltpu.CompilerParams(
            collective_id=0,
            vmem_limit_bytes=64 * 1024 * 1024,
        ),
    )(q, k_t, v_t, wo)

    return out[None]


# device time: 618139 ns/iter; 1.1484x vs baseline; 1.0319x over previous
import functools

import jax
import jax.numpy as jnp
from jax import lax
from jax.experimental import pallas as pl
from jax.experimental.pallas import tpu as pltpu

N_DEV = 4
SQ = 2048
SKV_SHARD = 2048
HQ_LOCAL = 8
DH = 128
D_MODEL = 1024
BLK = 64
SCALE = 0.08838834764831843
QT = 512
N_QT = SQ // QT
QUARTER = SQ // N_DEV

f32 = jnp.float32
bf16 = jnp.bfloat16


def _mask_tile(coff, t):
    toff = (t * QT).astype(f32)
    qi = lax.broadcasted_iota(jnp.int32, (QT, SKV_SHARD), 0).astype(f32) + toff
    ki = lax.broadcasted_iota(jnp.int32, (QT, SKV_SHARD), 1).astype(f32)
    qb = jnp.floor(qi * (1.0 / BLK))
    kb = jnp.floor(ki * (1.0 / BLK)) + coff
    sm = qb + kb
    mod3 = sm - 3.0 * jnp.floor(sm * (1.0 / 3.0))
    keep = (qb == kb) | (kb == 0.0) | (mod3 == 0.0)
    return keep


def kernel(x, Wq, K_ext, V_ext, Wo):
    x2 = x[0].astype(bf16)
    q = jnp.dot(x2, Wq.astype(bf16), preferred_element_type=f32)
    q = (q * SCALE).astype(bf16)
    k_t = jnp.transpose(K_ext[0].astype(bf16), (1, 0, 2))
    v_t = jnp.transpose(V_ext[0].astype(bf16), (1, 0, 2))
    wo = Wo.astype(bf16)

    def body(
        q_ref,
        kt_ref,
        vt_ref,
        wo_ref,
        out_ref,
        k_hbm,
        v_hbm,
        k_tile,
        v_tile,
        acc_ref,
        l_ref,
        mask_ref,
        rs_ref,
        pb_ref,
        agq_ref,
        agr_ref,
        ksend_sems, vsend_sems,
        krecv_sems, vrecv_sems,
        lc_sems,
        ktile_sems, vtile_sems,
        rssend_sems, rsrecv_sems,
        agsend_sems, agrecv_sems,
    ):
        me = lax.axis_index("i")

        barrier = pltpu.get_barrier_semaphore()
        for dj in range(1, N_DEV):
            pl.semaphore_signal(
                barrier, inc=1,
                device_id=((me + dj) % N_DEV,),
                device_id_type=pl.DeviceIdType.MESH,
            )
        pl.semaphore_wait(barrier, N_DEV - 1)

        sends = []
        for dj in range(1, N_DEV):
            tgt = (me + dj) % N_DEV
            for (src, dst, ssem, rsem) in (
                (kt_ref, k_hbm, ksend_sems, krecv_sems),
                (vt_ref, v_hbm, vsend_sems, vrecv_sems),
            ):
                rdma = pltpu.make_async_remote_copy(
                    src_ref=src.at[pl.ds(HQ_LOCAL * tgt, HQ_LOCAL)],
                    dst_ref=dst.at[me],
                    send_sem=ssem.at[dj - 1],
                    recv_sem=rsem.at[dj - 1],
                    device_id=(tgt,),
                    device_id_type=pl.DeviceIdType.MESH,
                )
                rdma.start()
                sends.append(rdma)

        locals_ = []
        for i, (src, dst) in enumerate(((kt_ref, k_hbm), (vt_ref, v_hbm))):
            cp = pltpu.make_async_copy(
                src.at[pl.ds(HQ_LOCAL * me, HQ_LOCAL)],
                dst.at[me],
                lc_sems.at[i],
            )
            cp.start()
            locals_.append(cp)

        def wait_chunk(dc):
            if dc == 0:
                for cp in locals_:
                    cp.wait()
                return
            sl = 3 - dc
            pltpu.make_async_remote_copy(
                src_ref=kt_ref.at[pl.ds(0, HQ_LOCAL)], dst_ref=k_hbm.at[0],
                send_sem=ksend_sems.at[sl], recv_sem=krecv_sems.at[sl],
                device_id=(0,), device_id_type=pl.DeviceIdType.MESH,
            ).wait_recv()
            pltpu.make_async_remote_copy(
                src_ref=vt_ref.at[pl.ds(0, HQ_LOCAL)], dst_ref=v_hbm.at[0],
                send_sem=vsend_sems.at[sl], recv_sem=vrecv_sems.at[sl],
                device_id=(0,), device_id_type=pl.DeviceIdType.MESH,
            ).wait_recv()

        l_ref[...] = jnp.zeros((HQ_LOCAL, SQ, DH), f32)
        acc_ref[...] = jnp.zeros((HQ_LOCAL, SQ, DH), f32)

        def tile_copies(cv, h, slot):
            kc = pltpu.make_async_copy(
                k_hbm.at[cv, h], k_tile.at[slot], ktile_sems.at[slot])
            vc = pltpu.make_async_copy(
                v_hbm.at[cv, h], v_tile.at[slot], vtile_sems.at[slot])
            return kc, vc

        DCS = [0, 1, 3, 2]
        cvals = [(me + dc) % N_DEV for dc in DCS]
        items = [(ci, h) for ci in range(N_DEV) for h in range(HQ_LOCAL)]
        inflight = {}
        wait_chunk(DCS[0])
        kc, vc = tile_copies(cvals[0], 0, 0)
        kc.start(); vc.start()
        inflight[0] = (kc, vc)

        for idx, (ci, h) in enumerate(items):
            if idx + 1 < len(items):
                nci, nh = items[idx + 1]
                if nh == 0:
                    wait_chunk(DCS[nci])
                slot = (idx + 1) % 2
                kc, vc = tile_copies(cvals[nci], nh, slot)
                kc.start(); vc.start()
                inflight[idx + 1] = (kc, vc)
            kc, vc = inflight.pop(idx)
            kc.wait(); vc.wait()
            slot = idx % 2

            if h == 0:
                coff = (cvals[ci] * (SKV_SHARD // BLK)).astype(f32)

                def mask_body(t, _, coff=coff):
                    mask_ref[pl.ds(t * QT, QT), :] = (
                        _mask_tile(coff, t).astype(jnp.int8))
                    return _
                lax.fori_loop(0, N_QT, mask_body, None, unroll=False)

            k = k_tile[slot]
            v = v_tile[slot]

            def tile_body(t, _):
                rows = pl.ds(t * QT, QT)
                s = lax.dot_general(
                    q_ref[rows, h * DH:(h + 1) * DH],
                    k, (((1,), (1,)), ((), ())),
                    preferred_element_type=f32,
                )
                p = jnp.exp(s) * mask_ref[rows, :].astype(f32)
                l_ref[h, rows, :] += jnp.sum(p, axis=1, keepdims=True)
                acc_ref[h, rows, :] += lax.dot_general(
                    p.astype(bf16), v, (((1,), (0,)), ((), ())),
                    preferred_element_type=f32,
                )
                return _
            lax.fori_loop(0, N_QT, tile_body, None, unroll=2)

        rs_sends = []
        for dj in range(1, N_DEV + 1):
            tgt = (me + dj) % N_DEV
            rows = pl.ds(QUARTER * tgt, QUARTER)
            part = jnp.zeros((QUARTER, D_MODEL), f32)
            for h in range(HQ_LOCAL):
                ctx_h = (acc_ref[h, rows, :] / l_ref[h, rows, :]).astype(bf16)
                part = part + lax.dot_general(
                    ctx_h, wo_ref[h * DH:(h + 1) * DH, :],
                    (((1,), (0,)), ((), ())), preferred_element_type=f32)
            out_ref[rows, :] = part
            if dj < N_DEV:
                pb_ref[rows, :] = part.astype(bf16)
                rdma = pltpu.make_async_remote_copy(
                    src_ref=pb_ref.at[rows, :],
                    dst_ref=rs_ref.at[dj - 1],
                    send_sem=rssend_sems.at[dj - 1],
                    recv_sem=rsrecv_sems.at[dj - 1],
                    device_id=(tgt,),
                    device_id_type=pl.DeviceIdType.MESH,
                )
                rdma.start()
                rs_sends.append(rdma)
        for dj in range(1, N_DEV):
            pltpu.make_async_remote_copy(
                src_ref=pb_ref.at[pl.ds(0, QUARTER), :],
                dst_ref=rs_ref.at[dj - 1],
                send_sem=rssend_sems.at[dj - 1],
                recv_sem=rsrecv_sems.at[dj - 1],
                device_id=(0,), device_id_type=pl.DeviceIdType.MESH,
            ).wait_recv()
        mine = out_ref[pl.ds(QUARTER * me, QUARTER), :]
        mine = (mine + rs_ref[0].astype(f32) + rs_ref[1].astype(f32)
                + rs_ref[2].astype(f32))
        out_ref[pl.ds(QUARTER * me, QUARTER), :] = mine
        agq_ref[...] = mine.astype(bf16)
        for r in rs_sends:
            r.wait_send()

        ag_sends = []
        for dj in range(1, N_DEV):
            tgt = (me + dj) % N_DEV
            rdma = pltpu.make_async_remote_copy(
                src_ref=agq_ref,
                dst_ref=agr_ref.at[dj - 1],
                send_sem=agsend_sems.at[dj - 1],
                recv_sem=agrecv_sems.at[dj - 1],
                device_id=(tgt,),
                device_id_type=pl.DeviceIdType.MESH,
            )
            rdma.start()
            ag_sends.append(rdma)
        for dj in range(1, N_DEV):
            pltpu.make_async_remote_copy(
                src_ref=agq_ref,
                dst_ref=agr_ref.at[dj - 1],
                send_sem=agsend_sems.at[dj - 1],
                recv_sem=agrecv_sems.at[dj - 1],
                device_id=(0,), device_id_type=pl.DeviceIdType.MESH,
            ).wait_recv()
        for sl in range(N_DEV - 1):
            src_dev = (me - sl - 1) % N_DEV
            out_ref[pl.ds(QUARTER * src_dev, QUARTER), :] = (
                agr_ref[sl].astype(f32))

        for r in sends + ag_sends:
            r.wait_send()

    out, _, _ = pl.pallas_call(
        body,
        out_shape=(
            jax.ShapeDtypeStruct((SQ, D_MODEL), f32),
            jax.ShapeDtypeStruct((N_DEV, HQ_LOCAL, SKV_SHARD, DH), bf16),
            jax.ShapeDtypeStruct((N_DEV, HQ_LOCAL, SKV_SHARD, DH), bf16),
        ),
        in_specs=[
            pl.BlockSpec(memory_space=pltpu.MemorySpace.VMEM),
            pl.BlockSpec(memory_space=pl.ANY),
            pl.BlockSpec(memory_space=pl.ANY),
            pl.BlockSpec(memory_space=pltpu.MemorySpace.VMEM),
        ],
        out_specs=(
            pl.BlockSpec(memory_space=pltpu.MemorySpace.VMEM),
            pl.BlockSpec(memory_space=pl.ANY),
            pl.BlockSpec(memory_space=pl.ANY),
        ),
        scratch_shapes=[
            pltpu.VMEM((2, SKV_SHARD, DH), bf16),
            pltpu.VMEM((2, SKV_SHARD, DH), bf16),
            pltpu.VMEM((HQ_LOCAL, SQ, DH), f32),
            pltpu.VMEM((HQ_LOCAL, SQ, DH), f32),
            pltpu.VMEM((SQ, SKV_SHARD), jnp.int8),
            pltpu.VMEM((3, QUARTER, D_MODEL), bf16),
            pltpu.VMEM((SQ, D_MODEL), bf16),
            pltpu.VMEM((QUARTER, D_MODEL), bf16),
            pltpu.VMEM((3, QUARTER, D_MODEL), bf16),
            pltpu.SemaphoreType.DMA((3,)),
            pltpu.SemaphoreType.DMA((3,)),
            pltpu.SemaphoreType.DMA((3,)),
            pltpu.SemaphoreType.DMA((3,)),
            pltpu.SemaphoreType.DMA((2,)),
            pltpu.SemaphoreType.DMA((2,)),
            pltpu.SemaphoreType.DMA((2,)),
            pltpu.SemaphoreType.DMA((3,)),
            pltpu.SemaphoreType.DMA((3,)),
            pltpu.SemaphoreType.DMA((3,)),
            pltpu.SemaphoreType.DMA((3,)),
        ],
        compiler_params=pltpu.CompilerParams(
            collective_id=0,
            vmem_limit_bytes=64 * 1024 * 1024,
        ),
    )(q, k_t, v_t, wo)

    return out[None]
